# baseline (speedup 1.0000x reference)
import functools

import jax
import jax.numpy as jnp
from jax import lax
from jax.experimental import pallas as pl
from jax.experimental.pallas import tpu as pltpu
from jax.experimental.pallas import tpu_sc as plsc

B = 16384
D = 32

_info = plsc.get_sparse_core_info()
_NC, _NS = _info.num_cores, _info.num_subcores
NW = _NC * _NS
BPW = B // NW
IC = 128
CH = BPW // IC


def _sc_gather(uidx2d, iidx2d, ue_gmf, ie_gmf, ue_mlp, ie_mlp):
    mesh = plsc.VectorSubcoreMesh(core_axis_name="c", subcore_axis_name="s")

    @functools.partial(
        pl.kernel,
        mesh=mesh,
        compiler_params=pltpu.CompilerParams(use_tc_tiling_on_sc=False),
        out_type=[jax.ShapeDtypeStruct((B, D), jnp.float32) for _ in range(4)],
        scratch_types=[
            pltpu.VMEM((CH, IC), jnp.int32),
            pltpu.VMEM((CH, IC), jnp.int32),
            pltpu.VMEM((BPW, D), jnp.float32),
            pltpu.VMEM((BPW, D), jnp.float32),
            pltpu.VMEM((BPW, D), jnp.float32),
            pltpu.VMEM((BPW, D), jnp.float32),
            pltpu.SemaphoreType.DMA,
        ],
    )
    def k(uidx_hbm, iidx_hbm, ug_hbm, ig_hbm, um_hbm, im_hbm,
          ug_out, ig_out, um_out, im_out,
          uidx_v, iidx_v, ug_v, ig_v, um_v, im_v, sem):
        wid = lax.axis_index("s") * _NC + lax.axis_index("c")
        base = wid * BPW
        row0 = wid * CH
        pltpu.sync_copy(uidx_hbm.at[pl.ds(row0, CH)], uidx_v)
        pltpu.sync_copy(iidx_hbm.at[pl.ds(row0, CH)], iidx_v)
        descs = []
        for c in range(CH):
            lo = c * IC
            descs.append(pltpu.async_copy(
                ug_hbm.at[uidx_v.at[c]], ug_v.at[pl.ds(lo, IC)], sem))
            descs.append(pltpu.async_copy(
                ig_hbm.at[iidx_v.at[c]], ig_v.at[pl.ds(lo, IC)], sem))
            descs.append(pltpu.async_copy(
                um_hbm.at[uidx_v.at[c]], um_v.at[pl.ds(lo, IC)], sem))
            descs.append(pltpu.async_copy(
                im_hbm.at[iidx_v.at[c]], im_v.at[pl.ds(lo, IC)], sem))
        for dsc in descs:
            dsc.wait()
        pltpu.sync_copy(ug_v, ug_out.at[pl.ds(base, BPW)])
        pltpu.sync_copy(ig_v, ig_out.at[pl.ds(base, BPW)])
        pltpu.sync_copy(um_v, um_out.at[pl.ds(base, BPW)])
        pltpu.sync_copy(im_v, im_out.at[pl.ds(base, BPW)])

    return k(uidx2d, iidx2d, ue_gmf, ie_gmf, ue_mlp, ie_mlp)


def _tc_dense(ug, ig, um, im, w0u, w0i, b0, w1t, b1, w2t, b2, w3t, b3,
              wpg, wph, bp):
    TM = 2048

    def body(ug_r, ig_r, um_r, im_r, w0u_r, w0i_r, b0_r, w1_r, b1_r,
             w2_r, b2_r, w3_r, b3_r, wpg_r, wph_r, bp_r, out_r):
        dot = functools.partial(jnp.dot, preferred_element_type=jnp.float32)
        h = dot(um_r[...], w0u_r[...]) + dot(im_r[...], w0i_r[...]) + b0_r[...]
        h = jnp.maximum(h, 0.0)
        h = jnp.maximum(dot(h, w1_r[...]) + b1_r[...], 0.0)
        h = jnp.maximum(dot(h, w2_r[...]) + b2_r[...], 0.0)
        h = jnp.maximum(dot(h, w3_r[...]) + b3_r[...], 0.0)
        logit = (dot(ug_r[...] * ig_r[...], wpg_r[...])
                 + dot(h, wph_r[...]) + bp_r[...])
        out_r[...] = 1.0 / (1.0 + jnp.exp(-logit))

    data_spec = pl.BlockSpec((TM, D), lambda i: (i, 0))
    full = lambda a: pl.BlockSpec(a.shape, lambda i: (0, 0))
    return pl.pallas_call(
        body,
        grid=(B // TM,),
        in_specs=[data_spec, data_spec, data_spec, data_spec,
                  full(w0u), full(w0i), full(b0), full(w1t), full(b1),
                  full(w2t), full(b2), full(w3t), full(b3),
                  full(wpg), full(wph), full(bp)],
        out_specs=pl.BlockSpec((TM, 1), lambda i: (i, 0)),
        out_shape=jax.ShapeDtypeStruct((B, 1), jnp.float32),
    )(ug, ig, um, im, w0u, w0i, b0, w1t, b1, w2t, b2, w3t, b3, wpg, wph, bp)


def kernel(user_indices, item_indices, ue_gmf, ie_gmf, ue_mlp, ie_mlp,
           W0, b0, W1, b1, W2, b2, W3, b3, Wp, bp):
    uidx2d = user_indices.astype(jnp.int32).reshape(B // IC, IC)
    iidx2d = item_indices.astype(jnp.int32).reshape(B // IC, IC)
    ug, ig, um, im = _sc_gather(uidx2d, iidx2d, ue_gmf, ie_gmf, ue_mlp, ie_mlp)
    w0u = W0[:, :D].T
    w0i = W0[:, D:].T
    wpg = Wp[:, :D].T
    wph = Wp[:, D:].T
    return _tc_dense(ug, ig, um, im, w0u, w0i, b0.reshape(1, -1),
                     W1.T, b1.reshape(1, -1), W2.T, b2.reshape(1, -1),
                     W3.T, b3.reshape(1, -1), wpg, wph, bp.reshape(1, 1))
